# baseline (speedup 1.0000x reference)
import functools

import jax
import jax.numpy as jnp
from jax import lax
from jax.experimental import pallas as pl
from jax.experimental.pallas import tpu as pltpu
from jax.experimental.pallas import tpu_sc as plsc

LANES = 16
NW = 32
CHUNK = 128
PAD = LANES + 1


def _tree_sum(xs):
    while len(xs) > 1:
        xs = [a + b for a, b in zip(xs[0::2], xs[1::2])]
    return xs[0]


def _qtable_body(bpw, d, state_hbm, action_hbm, w_hbm, out_hbm,
                 idx_v, w_v0, w_v1, w_v2, s_v0, s_v1, s_v2, out_v, m_v,
                 sem0, sem1, sem2):
    wid = lax.axis_index("s") * 2 + lax.axis_index("c")
    base = wid * bpw
    nf = d // LANES
    nch = bpw // CHUNK
    wbufs, sbufs = [w_v0, w_v1, w_v2], [s_v0, s_v1, s_v2]
    sems = [sem0, sem1, sem2]
    nbuf = 3

    pltpu.sync_copy(action_hbm.at[pl.ds(base, bpw)], idx_v)

    cps = {}

    def start(ch):
        b = ch % nbuf
        cps[ch] = [
            pltpu.async_copy(
                w_hbm.at[idx_v.at[pl.ds(ch * CHUNK, CHUNK)]], wbufs[b], sems[b]),
            pltpu.async_copy(
                state_hbm.at[pl.ds(base + ch * CHUNK, CHUNK)], sbufs[b], sems[b]),
        ]

    lanes17 = lax.iota(jnp.int32, LANES) * PAD
    start(0)
    start(1)
    for ch in range(nch):
        if ch + 2 < nch:
            start(ch + 2)
        for cp in cps.pop(ch):
            cp.wait()
        wv, sv = wbufs[ch % nbuf], sbufs[ch % nbuf]

        def group(g, _, wv=wv, sv=sv, ch=ch):
            for j in range(LANES):
                row = g * LANES + j
                ps = [wv[row, pl.ds(k * LANES, LANES)]
                      * sv[row, pl.ds(k * LANES, LANES)] for k in range(nf)]
                m_v[pl.ds(j * PAD, LANES)] = _tree_sum(ps)
            cols = [plsc.load_gather(m_v, [lanes17 + l]) for l in range(LANES)]
            out_v[pl.ds(ch * CHUNK + g * LANES, LANES)] = _tree_sum(cols)
            return 0

        lax.fori_loop(0, CHUNK // LANES, group, 0)

    pltpu.sync_copy(out_v, out_hbm.at[pl.ds(base, bpw)])


@jax.jit
def kernel(state, action, weights):
    b = state.shape[0]
    d = state.shape[1] * state.shape[2]
    actions = weights.shape[0]
    bpw = b // NW

    state2 = state.reshape(b, d)
    weights2 = weights.reshape(actions, d)
    action32 = action.astype(jnp.int32)

    mesh = plsc.VectorSubcoreMesh(core_axis_name="c", subcore_axis_name="s")
    f = pl.kernel(
        functools.partial(_qtable_body, bpw, d),
        mesh=mesh,
        out_type=jax.ShapeDtypeStruct((b,), jnp.float32),
        compiler_params=pltpu.CompilerParams(needs_layout_passes=False),
        scratch_types=[
            pltpu.VMEM((bpw,), jnp.int32),
            pltpu.VMEM((CHUNK, d), jnp.float32),
            pltpu.VMEM((CHUNK, d), jnp.float32),
            pltpu.VMEM((CHUNK, d), jnp.float32),
            pltpu.VMEM((CHUNK, d), jnp.float32),
            pltpu.VMEM((CHUNK, d), jnp.float32),
            pltpu.VMEM((CHUNK, d), jnp.float32),
            pltpu.VMEM((bpw,), jnp.float32),
            pltpu.VMEM((LANES * PAD,), jnp.float32),
            pltpu.SemaphoreType.DMA,
            pltpu.SemaphoreType.DMA,
            pltpu.SemaphoreType.DMA,
        ],
    )
    return f(state2, action32, weights2)


# speedup vs baseline: 1.0166x; 1.0166x over previous
import functools

import jax
import jax.numpy as jnp
from jax import lax
from jax.experimental import pallas as pl
from jax.experimental.pallas import tpu as pltpu
from jax.experimental.pallas import tpu_sc as plsc

LANES = 16
NW = 32
CHUNK = 128
PAD = LANES + 1


def _tree_sum(xs):
    while len(xs) > 1:
        xs = [a + b for a, b in zip(xs[0::2], xs[1::2])]
    return xs[0]


def _qtable_body(bpw, d, state_hbm, action_hbm, w_hbm, out_hbm,
                 idx_v, w_v0, w_v1, s_v0, s_v1, out_v, m_v, sem0, sem1):
    wid = lax.axis_index("s") * 2 + lax.axis_index("c")
    base = wid * bpw
    nf = d // LANES
    nch = bpw // CHUNK
    wbufs, sbufs, sems = [w_v0, w_v1], [s_v0, s_v1], [sem0, sem1]

    pltpu.sync_copy(action_hbm.at[pl.ds(base, CHUNK)],
                    idx_v.at[pl.ds(0, CHUNK)])

    cps = {}

    def start(ch):
        b = ch % 2
        cps[ch] = [
            pltpu.async_copy(
                w_hbm.at[idx_v.at[pl.ds(ch * CHUNK, CHUNK)]], wbufs[b], sems[b]),
            pltpu.async_copy(
                state_hbm.at[pl.ds(base + ch * CHUNK, CHUNK)], sbufs[b], sems[b]),
        ]

    lanes17 = lax.iota(jnp.int32, LANES) * PAD
    start(0)
    pltpu.sync_copy(action_hbm.at[pl.ds(base + CHUNK, bpw - CHUNK)],
                    idx_v.at[pl.ds(CHUNK, bpw - CHUNK)])
    for ch in range(nch):
        if ch + 1 < nch:
            start(ch + 1)
        for cp in cps.pop(ch):
            cp.wait()
        wv, sv = wbufs[ch % 2], sbufs[ch % 2]

        def group(g, _, wv=wv, sv=sv, ch=ch):
            for j in range(LANES):
                row = g * LANES + j
                ps = [wv[row, pl.ds(k * LANES, LANES)]
                      * sv[row, pl.ds(k * LANES, LANES)] for k in range(nf)]
                m_v[pl.ds(j * PAD, LANES)] = _tree_sum(ps)
            cols = [plsc.load_gather(m_v, [lanes17 + l]) for l in range(LANES)]
            out_v[pl.ds(ch * CHUNK + g * LANES, LANES)] = _tree_sum(cols)
            return 0

        lax.fori_loop(0, CHUNK // LANES, group, 0)

    pltpu.sync_copy(out_v, out_hbm.at[pl.ds(base, bpw)])


@jax.jit
def kernel(state, action, weights):
    b = state.shape[0]
    d = state.shape[1] * state.shape[2]
    actions = weights.shape[0]
    bpw = b // NW

    state2 = state.reshape(b, d)
    weights2 = weights.reshape(actions, d)
    action32 = action.astype(jnp.int32)

    mesh = plsc.VectorSubcoreMesh(core_axis_name="c", subcore_axis_name="s")
    f = pl.kernel(
        functools.partial(_qtable_body, bpw, d),
        mesh=mesh,
        out_type=jax.ShapeDtypeStruct((b,), jnp.float32),
        compiler_params=pltpu.CompilerParams(needs_layout_passes=False),
        scratch_types=[
            pltpu.VMEM((bpw,), jnp.int32),
            pltpu.VMEM((CHUNK, d), jnp.float32),
            pltpu.VMEM((CHUNK, d), jnp.float32),
            pltpu.VMEM((CHUNK, d), jnp.float32),
            pltpu.VMEM((CHUNK, d), jnp.float32),
            pltpu.VMEM((bpw,), jnp.float32),
            pltpu.VMEM((LANES * PAD,), jnp.float32),
            pltpu.SemaphoreType.DMA,
            pltpu.SemaphoreType.DMA,
        ],
    )
    return f(state2, action32, weights2)
